# baseline (speedup 1.0000x reference)
import functools

import jax
import jax.numpy as jnp
from jax import lax
from jax.experimental import pallas as pl
from jax.experimental.pallas import tpu as pltpu
from jax.experimental.pallas import tpu_sc as plsc

NC = 2
NS = 16
NW = NC * NS
BATCH = 16384
DIM = 64
BPW = BATCH // NW
CH = 128
NCH = BPW // CH


def _body(users_hbm, items_hbm, utab_hbm, itab_hbm, uout_hbm, iout_hbm,
          uidx_v, iidx_v, urows_v, irows_v, gsem, osem):
    wid = lax.axis_index("s") * NC + lax.axis_index("c")
    base = wid * BPW
    row = wid * NCH
    pltpu.sync_copy(users_hbm.at[pl.ds(row, NCH)], uidx_v)
    pltpu.sync_copy(items_hbm.at[pl.ds(row, NCH)], iidx_v)
    copies = []
    for j in range(NCH):
        copies.append(pltpu.async_copy(
            utab_hbm.at[uidx_v.at[j]], urows_v.at[pl.ds(j * CH, CH)], gsem))
    for j in range(NCH):
        copies.append(pltpu.async_copy(
            itab_hbm.at[iidx_v.at[j]], irows_v.at[pl.ds(j * CH, CH)], gsem))
    for c in copies[:NCH]:
        c.wait()
    out_u = pltpu.async_copy(urows_v, uout_hbm.at[pl.ds(base, BPW)], osem)
    for c in copies[NCH:]:
        c.wait()
    out_i = pltpu.async_copy(irows_v, iout_hbm.at[pl.ds(base, BPW)], osem)
    out_u.wait()
    out_i.wait()


@functools.cache
def _build():
    mesh = plsc.VectorSubcoreMesh(core_axis_name="c", subcore_axis_name="s",
                                  num_cores=NC, num_subcores=NS)
    return pl.kernel(
        _body,
        out_type=(jax.ShapeDtypeStruct((BATCH, DIM), jnp.float32),
                  jax.ShapeDtypeStruct((BATCH, DIM), jnp.float32)),
        mesh=mesh,
        scratch_types=[
            pltpu.VMEM((NCH, CH), jnp.int32),
            pltpu.VMEM((NCH, CH), jnp.int32),
            pltpu.VMEM((BPW, DIM), jnp.float32),
            pltpu.VMEM((BPW, DIM), jnp.float32),
            pltpu.SemaphoreType.DMA,
            pltpu.SemaphoreType.DMA,
        ],
        compiler_params=pltpu.CompilerParams(use_tc_tiling_on_sc=False),
    )


@jax.jit
def kernel(users, items, user_table, item_table):
    users2d = jnp.asarray(users, jnp.int32).reshape(BATCH // CH, CH)
    items2d = jnp.asarray(items, jnp.int32).reshape(BATCH // CH, CH)
    return _build()(users2d, items2d, user_table, item_table)


# speedup vs baseline: 1.1051x; 1.1051x over previous
import functools

import jax
import jax.numpy as jnp
from jax import lax
from jax.experimental import pallas as pl
from jax.experimental.pallas import tpu as pltpu
from jax.experimental.pallas import tpu_sc as plsc

NC = 2
NS = 16
NW = NC * NS
BATCH = 16384
DIM = 64
PDIM = 128
BPW = BATCH // NW
CH = 128
NCH = BPW // CH
NCHUNKS = 2 * NCH
NBUF = 4


def _body(users_hbm, items_hbm, utab_hbm, itab_hbm, uout_hbm, iout_hbm,
          uidx_v, iidx_v, b0, b1, b2, b3, gs0, gs1, gs2, gs3,
          os0, os1, os2, os3):
    bufs = [b0, b1, b2, b3]
    gsems = [gs0, gs1, gs2, gs3]
    osems = [os0, os1, os2, os3]
    wid = lax.axis_index("s") * NC + lax.axis_index("c")
    base = wid * BPW
    row = wid * NCH
    pltpu.sync_copy(users_hbm.at[pl.ds(row, NCH)], uidx_v)
    pltpu.sync_copy(items_hbm.at[pl.ds(row, NCH)], iidx_v)

    def gather(c, b):
        tab = utab_hbm if c < NCH else itab_hbm
        idx = uidx_v if c < NCH else iidx_v
        return pltpu.async_copy(tab.at[idx.at[c % NCH]], bufs[b], gsems[b])

    def out(c):
        b = c % NBUF
        dst = uout_hbm if c < NCH else iout_hbm
        return pltpu.async_copy(
            bufs[b], dst.at[pl.ds(base + (c % NCH) * CH, CH)], osems[b])

    g_h = [None] * NCHUNKS
    o_h = [None] * NCHUNKS
    for c in range(NCHUNKS):
        if c >= NBUF:
            o_h[c - NBUF].wait()
        g_h[c] = gather(c, c % NBUF)
        if c >= 2:
            g_h[c - 2].wait()
            o_h[c - 2] = out(c - 2)
    for c in range(NCHUNKS - 2, NCHUNKS):
        g_h[c].wait()
        o_h[c] = out(c)
    for c in range(NCHUNKS - NBUF, NCHUNKS):
        o_h[c].wait()


@functools.cache
def _build():
    mesh = plsc.VectorSubcoreMesh(core_axis_name="c", subcore_axis_name="s",
                                  num_cores=NC, num_subcores=NS)
    return pl.kernel(
        _body,
        out_type=(jax.ShapeDtypeStruct((BATCH, PDIM), jnp.float32),
                  jax.ShapeDtypeStruct((BATCH, PDIM), jnp.float32)),
        mesh=mesh,
        scratch_types=[
            pltpu.VMEM((NCH, CH), jnp.int32),
            pltpu.VMEM((NCH, CH), jnp.int32),
            *[pltpu.VMEM((CH, PDIM), jnp.float32) for _ in range(NBUF)],
            *[pltpu.SemaphoreType.DMA for _ in range(2 * NBUF)],
        ],
    )


@jax.jit
def kernel(users, items, user_table, item_table):
    users2d = jnp.asarray(users, jnp.int32).reshape(BATCH // CH, CH)
    items2d = jnp.asarray(items, jnp.int32).reshape(BATCH // CH, CH)
    utab = jnp.pad(user_table, ((0, 0), (0, PDIM - DIM)))
    itab = jnp.pad(item_table, ((0, 0), (0, PDIM - DIM)))
    uout, iout = _build()(users2d, items2d, utab, itab)
    return uout[:, :DIM], iout[:, :DIM]
